# baseline (speedup 1.0000x reference)
import jax
import jax.numpy as jnp
from jax.experimental import pallas as pl
from jax.experimental.pallas import tpu as pltpu

S_BLK = 512
R = 4


def _body(lens_ref, x0_ref, x1_ref, x2_ref, x3_ref, w_ref, o_ref,
          ml_ref, acc_ref):
    g = pl.program_id(0)
    j = pl.program_id(1)
    xrefs = (x0_ref, x1_ref, x2_ref, x3_ref)

    for r in range(R):
        row = g * R + r
        length = lens_ref[row]
        last = (length - 1) // S_BLK
        x_ref = xrefs[r]

        def _update(masked, r=r, x_ref=x_ref, length=length):
            xb = x_ref[...]
            logits = jax.lax.dot_general(
                xb, w_ref[...], (((1,), (0,)), ((), ())),
                preferred_element_type=jnp.float32)
            if masked:
                pos = j * S_BLK + jax.lax.broadcasted_iota(
                    jnp.int32, (S_BLK, 1), 0)
                logits = jnp.where(pos < length, logits, -jnp.inf)
            m_prev = jnp.where(j == 0, -jnp.inf, ml_ref[0, r])
            l_prev = jnp.where(j == 0, 0.0, ml_ref[1, r])
            m_new = jnp.maximum(m_prev, jnp.max(logits))
            alpha = jnp.exp(m_prev - m_new)
            p = jnp.exp(logits - m_new)
            ml_ref[0, r] = m_new
            l_new = l_prev * alpha + jnp.sum(p)
            ml_ref[1, r] = l_new
            px = jax.lax.dot_general(
                p, xb, (((0,), (0,)), ((), ())),
                preferred_element_type=jnp.float32)
            prev = jnp.where(j == 0, jnp.zeros_like(acc_ref[r:r + 1]),
                             acc_ref[r:r + 1])
            acc = prev * alpha + px
            if masked:
                o_ref[0, r:r + 1] = acc / l_new
            acc_ref[r:r + 1] = acc

        @pl.when(j < last)
        def _interior():
            _update(masked=False)

        @pl.when(j == last)
        def _final():
            _update(masked=True)


def kernel(x, x_lens, W, b):
    B, S, D = x.shape
    nblk = S // S_BLK
    lens = x_lens.astype(jnp.int32)
    x2 = x.reshape(B * S, D)

    def _xmap(r):
        def im(g, j, lens):
            row = g * R + r
            return (row * nblk + jnp.minimum(j, (lens[row] - 1) // S_BLK), 0)
        return im

    out = pl.pallas_call(
        _body,
        grid_spec=pltpu.PrefetchScalarGridSpec(
            num_scalar_prefetch=1,
            grid=(B // R, nblk),
            in_specs=[
                pl.BlockSpec((S_BLK, D), _xmap(0)),
                pl.BlockSpec((S_BLK, D), _xmap(1)),
                pl.BlockSpec((S_BLK, D), _xmap(2)),
                pl.BlockSpec((S_BLK, D), _xmap(3)),
                pl.BlockSpec((D, 1), lambda g, j, lens: (0, 0)),
            ],
            out_specs=pl.BlockSpec((1, R, D), lambda g, j, lens: (g, 0, 0)),
            scratch_shapes=[
                pltpu.SMEM((2, R), jnp.float32),
                pltpu.VMEM((R, D), jnp.float32),
            ],
        ),
        out_shape=jax.ShapeDtypeStruct((B // R, R, D), jnp.float32),
        compiler_params=pltpu.CompilerParams(
            dimension_semantics=("arbitrary", "arbitrary")),
    )(lens, x2, x2, x2, x2, W)
    return out.reshape(B, D)
